# baseline (speedup 1.0000x reference)
import functools

import jax
import jax.numpy as jnp
from jax import lax
from jax.experimental import pallas as pl
from jax.experimental.pallas import tpu as pltpu
from jax.experimental.pallas import tpu_sc as plsc

MARGIN = 0.25
GAMMA = 256.0

_NC = 2
_NS = 16
_NW = _NC * _NS
_CHUNK = 32000


def _transform(x):
    cos = jnp.minimum(jnp.maximum(x, -1.0), 1.0)
    alpha_n = jnp.maximum(cos + MARGIN, 0.0)
    return alpha_n * (cos - MARGIN) * GAMMA


def _make_sc_kernel(b, c):
    elems_per_w = (b * c) // _NW
    nchunk = elems_per_w // _CHUNK
    rows_per_w = b // _NW
    mesh = plsc.VectorSubcoreMesh(core_axis_name="c", subcore_axis_name="s")

    @functools.partial(
        pl.kernel,
        mesh=mesh,
        out_type=jax.ShapeDtypeStruct((b * c,), jnp.float32),
        scratch_types=[
            pltpu.VMEM((2, _CHUNK), jnp.float32),
            pltpu.VMEM((2, _CHUNK), jnp.float32),
            pltpu.VMEM((rows_per_w,), jnp.int32),
            pltpu.VMEM((rows_per_w,), jnp.int32),
            pltpu.VMEM((rows_per_w,), jnp.float32),
            pltpu.SemaphoreType.DMA((2,)),
            pltpu.SemaphoreType.DMA((2,)),
            pltpu.SemaphoreType.DMA,
        ],
    )
    def sc_kernel(cos_hbm, lab_hbm, out_hbm, inbuf, outbuf, labv, idxv, valv,
                  insem, outsem, fsem):
        wid = lax.axis_index("s") * _NC + lax.axis_index("c")
        base = wid * elems_per_w

        def start_in(k, slot):
            pltpu.make_async_copy(
                cos_hbm.at[pl.ds(base + k * _CHUNK, _CHUNK)],
                inbuf.at[slot],
                insem.at[slot],
            ).start()

        def wait_in(k, slot):
            pltpu.make_async_copy(
                cos_hbm.at[pl.ds(base + k * _CHUNK, _CHUNK)],
                inbuf.at[slot],
                insem.at[slot],
            ).wait()

        def start_out(k, slot):
            pltpu.make_async_copy(
                outbuf.at[slot],
                out_hbm.at[pl.ds(base + k * _CHUNK, _CHUNK)],
                outsem.at[slot],
            ).start()

        def wait_out(k, slot):
            pltpu.make_async_copy(
                outbuf.at[slot],
                out_hbm.at[pl.ds(base + k * _CHUNK, _CHUNK)],
                outsem.at[slot],
            ).wait()

        start_in(0, 0)
        start_in(1, 1)

        def chunk_body(k2, _):
            for slot in range(2):
                k = k2 * 2 + slot
                wait_in(k, slot)

                @pl.when(k2 > 0)
                def _():
                    wait_out(k - 2, slot)

                def vec_body(j, _):
                    x = inbuf[slot, pl.ds(j * 16, 16)]
                    outbuf[slot, pl.ds(j * 16, 16)] = _transform(x)
                    return _

                lax.fori_loop(0, _CHUNK // 16, vec_body, None, unroll=8)
                start_out(k, slot)

                @pl.when(k2 * 2 + slot + 2 < nchunk)
                def _():
                    start_in(k + 2, slot)
            return _

        lax.fori_loop(0, nchunk // 2, chunk_body, None)
        wait_out(nchunk - 2, 0)
        wait_out(nchunk - 1, 1)

        pltpu.sync_copy(lab_hbm.at[pl.ds(wid * rows_per_w, rows_per_w)], labv)
        for j in range(rows_per_w // 16):
            rows = lax.iota(jnp.int32, 16) + (wid * rows_per_w + j * 16)
            idxv[pl.ds(j * 16, 16)] = rows * c + labv[pl.ds(j * 16, 16)]
        pltpu.async_copy(cos_hbm.at[idxv], valv, fsem).wait()
        for j in range(rows_per_w // 16):
            g = valv[pl.ds(j * 16, 16)]
            valv[pl.ds(j * 16, 16)] = (
                jnp.minimum(jnp.maximum(g, -1.0), 1.0) * GAMMA
            )
        pltpu.async_copy(valv, out_hbm.at[idxv], fsem).wait()

    return sc_kernel


def kernel(cos_theta, labels):
    b, c = cos_theta.shape
    out_flat = _make_sc_kernel(b, c)(
        cos_theta.reshape(b * c), labels.astype(jnp.int32)
    )
    return out_flat.reshape(b, c)


# speedup vs baseline: 3.7588x; 3.7588x over previous
import jax
import jax.numpy as jnp
from jax.experimental import pallas as pl

MARGIN = 0.25
GAMMA = 256.0

_BLK_C = 2048


def _circle_loss_block(labels_ref, x_ref, o_ref):
    j = pl.program_id(0)
    x = x_ref[...]
    cos = jnp.clip(x, -1.0, 1.0)
    alpha_g = jnp.maximum(cos * GAMMA + (GAMMA * MARGIN), 0.0)
    neg = alpha_g * (cos - MARGIN)
    col = jax.lax.broadcasted_iota(jnp.int32, x.shape, 1) + j * _BLK_C
    mask = col == labels_ref[...]
    o_ref[...] = jnp.where(mask, cos * GAMMA, neg)


def kernel(cos_theta, labels):
    b, c = cos_theta.shape
    labels2d = labels.astype(jnp.int32).reshape(b, 1)
    grid = (pl.cdiv(c, _BLK_C),)
    return pl.pallas_call(
        _circle_loss_block,
        grid=grid,
        in_specs=[
            pl.BlockSpec((b, 1), lambda j: (0, 0)),
            pl.BlockSpec((b, _BLK_C), lambda j: (0, j)),
        ],
        out_specs=pl.BlockSpec((b, _BLK_C), lambda j: (0, j)),
        out_shape=jax.ShapeDtypeStruct((b, c), jnp.float32),
    )(labels2d, cos_theta)
